# Initial kernel scaffold; baseline (speedup 1.0000x reference)
import jax
import jax.numpy as jnp
from jax.experimental import pallas as pl


def kernel(relative_bias_table, relative_position_index):
    raise NotImplementedError("write your pallas kernel here")


# speedup vs baseline: 13.8765x; 13.8765x over previous
import functools

import jax
import jax.numpy as jnp
from jax import lax
from jax.experimental import pallas as pl
from jax.experimental.pallas import tpu as pltpu
from jax.experimental.pallas import tpu_sc as plsc

NHEADS = 16
NREL = 3969
TABLE_WORDS = NHEADS * NREL
TOTAL = 1024 * 1024
LANES = 16

_info = plsc.get_sparse_core_info()
NCORES = _info.num_cores
NSUB = _info.num_subcores
NWORKERS = NCORES * NSUB

PER_TILE = TOTAL // NWORKERS
CHUNK = 1024
NCHUNKS = PER_TILE // CHUNK


def _make_sc_gather():
    mesh = plsc.VectorSubcoreMesh(core_axis_name="c", subcore_axis_name="s")

    @functools.partial(
        pl.kernel,
        mesh=mesh,
        compiler_params=pltpu.CompilerParams(needs_layout_passes=False),
        out_type=jax.ShapeDtypeStruct((NHEADS, TOTAL), jnp.float32),
        scratch_types=[
            pltpu.VMEM((TABLE_WORDS,), jnp.float32),
            pltpu.VMEM((CHUNK,), jnp.int32),
            pltpu.VMEM((NHEADS, CHUNK), jnp.float32),
        ],
    )
    def gather_kernel(table_hbm, idx_hbm, out_hbm, table_v, idx_v, out_v):
        wid = lax.axis_index("s") * NCORES + lax.axis_index("c")
        base = wid * PER_TILE
        pltpu.sync_copy(table_hbm, table_v)

        def chunk_body(c, carry):
            cbase = base + c * CHUNK
            pltpu.sync_copy(idx_hbm.at[pl.ds(cbase, CHUNK)], idx_v)

            def gather_body(j, carry2):
                iv = idx_v[pl.ds(j * LANES, LANES)]
                for h in range(NHEADS):
                    off = iv + jnp.int32(h * NREL)
                    out_v[h, pl.ds(j * LANES, LANES)] = plsc.load_gather(
                        table_v, [off])
                return carry2

            lax.fori_loop(0, CHUNK // LANES, gather_body, 0)
            for h in range(NHEADS):
                pltpu.sync_copy(out_v.at[h], out_hbm.at[h, pl.ds(cbase, CHUNK)])
            return carry

        lax.fori_loop(0, NCHUNKS, chunk_body, 0)

    return gather_kernel


_sc_gather = _make_sc_gather()


def kernel(relative_bias_table, relative_position_index):
    num_heads = relative_bias_table.shape[0]
    side = relative_position_index.shape[0]
    table_flat = relative_bias_table.reshape(-1)
    idx_flat = relative_position_index.reshape(-1).astype(jnp.int32)
    out = _sc_gather(table_flat, idx_flat)
    return out.reshape(num_heads, side, side)


# speedup vs baseline: 18.0147x; 1.2982x over previous
import functools

import jax
import jax.numpy as jnp
from jax import lax
from jax.experimental import pallas as pl
from jax.experimental.pallas import tpu as pltpu
from jax.experimental.pallas import tpu_sc as plsc

NHEADS = 16
NREL = 3969
TABLE_WORDS = NHEADS * NREL
TOTAL = 1024 * 1024
LANES = 16

_info = plsc.get_sparse_core_info()
NCORES = _info.num_cores
NSUB = _info.num_subcores
NWORKERS = NCORES * NSUB

PER_TILE = TOTAL // NWORKERS
CHUNK = 1024
NCHUNKS = PER_TILE // CHUNK
NBUF = 2


def _make_sc_gather():
    mesh = plsc.VectorSubcoreMesh(core_axis_name="c", subcore_axis_name="s")

    @functools.partial(
        pl.kernel,
        mesh=mesh,
        compiler_params=pltpu.CompilerParams(needs_layout_passes=False),
        out_type=jax.ShapeDtypeStruct((NHEADS, TOTAL), jnp.float32),
        scratch_types=[
            pltpu.VMEM((TABLE_WORDS,), jnp.float32),
            pltpu.VMEM((NBUF, CHUNK), jnp.int32),
            pltpu.VMEM((NBUF, NHEADS, CHUNK), jnp.float32),
            pltpu.SemaphoreType.DMA,
            pltpu.SemaphoreType.DMA,
            pltpu.SemaphoreType.DMA,
        ],
    )
    def gather_kernel(table_hbm, idx_hbm, out_hbm, table_v, idx_v, out_v,
                      sem_idx, sem_out0, sem_out1):
        wid = lax.axis_index("s") * NCORES + lax.axis_index("c")
        base = wid * PER_TILE
        sem_out = (sem_out0, sem_out1)
        pltpu.sync_copy(table_hbm, table_v)

        pltpu.async_copy(idx_hbm.at[pl.ds(base, CHUNK)], idx_v.at[0], sem_idx)

        def outer(i, carry):
            for b in range(NBUF):
                c = i * NBUF + b
                cbase = base + c * CHUNK
                pltpu.make_async_copy(
                    idx_hbm.at[pl.ds(cbase, CHUNK)], idx_v.at[b],
                    sem_idx).wait()

                @pl.when(c + 1 < NCHUNKS)
                def _prefetch():
                    nbase = cbase + CHUNK
                    pltpu.async_copy(
                        idx_hbm.at[pl.ds(nbase, CHUNK)], idx_v.at[1 - b],
                        sem_idx)

                @pl.when(c >= NBUF)
                def _drain():
                    pbase = cbase - NBUF * CHUNK
                    pltpu.make_async_copy(
                        out_v.at[b], out_hbm.at[:, pl.ds(pbase, CHUNK)],
                        sem_out[b]).wait()

                def gather_body(j, carry2):
                    iv = idx_v[b, pl.ds(j * LANES, LANES)]
                    for h in range(NHEADS):
                        off = iv + jnp.int32(h * NREL)
                        out_v[b, h, pl.ds(j * LANES, LANES)] = (
                            plsc.load_gather(table_v, [off]))
                    return carry2

                lax.fori_loop(0, CHUNK // LANES, gather_body, 0)
                pltpu.async_copy(
                    out_v.at[b], out_hbm.at[:, pl.ds(cbase, CHUNK)],
                    sem_out[b])
            return carry

        lax.fori_loop(0, NCHUNKS // NBUF, outer, 0)

        for b in range(NBUF):
            cbase = base + (NCHUNKS - NBUF + b) * CHUNK
            pltpu.make_async_copy(
                out_v.at[b], out_hbm.at[:, pl.ds(cbase, CHUNK)],
                sem_out[b]).wait()

    return gather_kernel


_sc_gather = _make_sc_gather()


def kernel(relative_bias_table, relative_position_index):
    num_heads = relative_bias_table.shape[0]
    side = relative_position_index.shape[0]
    table_flat = relative_bias_table.reshape(-1)
    idx_flat = relative_position_index.reshape(-1).astype(jnp.int32)
    out = _sc_gather(table_flat, idx_flat)
    return out.reshape(num_heads, side, side)


# speedup vs baseline: 65.9214x; 3.6593x over previous
import functools

import jax
import jax.numpy as jnp
from jax import lax
from jax.experimental import pallas as pl
from jax.experimental.pallas import tpu as pltpu
from jax.experimental.pallas import tpu_sc as plsc

NHEADS = 16
NREL = 3969
TABLE_WORDS = NHEADS * NREL
SIDE = 1024
LANES = 16

_info = plsc.get_sparse_core_info()
NCORES = _info.num_cores
NSUB = _info.num_subcores
NWORKERS = NCORES * NSUB

ROWS_PER_TILE = SIDE // NWORKERS
NBUF = 2


def _make_sc_gather():
    mesh = plsc.VectorSubcoreMesh(core_axis_name="c", subcore_axis_name="s")

    @functools.partial(
        pl.kernel,
        mesh=mesh,
        compiler_params=pltpu.CompilerParams(needs_layout_passes=False),
        out_type=jax.ShapeDtypeStruct((NHEADS, SIDE, SIDE), jnp.float32),
        scratch_types=[
            pltpu.VMEM((TABLE_WORDS,), jnp.float32),
            pltpu.VMEM((NBUF, SIDE), jnp.int32),
            pltpu.VMEM((NBUF, NHEADS, SIDE), jnp.float32),
            pltpu.SemaphoreType.DMA,
            pltpu.SemaphoreType.DMA,
            pltpu.SemaphoreType.DMA,
        ],
    )
    def gather_kernel(table_hbm, idx_hbm, out_hbm, table_v, idx_v, out_v,
                      sem_idx, sem_out0, sem_out1):
        wid = lax.axis_index("s") * NCORES + lax.axis_index("c")
        row0 = wid * ROWS_PER_TILE
        sem_out = (sem_out0, sem_out1)
        pltpu.sync_copy(table_hbm, table_v)

        pltpu.async_copy(idx_hbm.at[row0, :], idx_v.at[0], sem_idx)

        def outer(i, carry):
            for b in range(NBUF):
                r = row0 + i * NBUF + b
                pltpu.make_async_copy(
                    idx_hbm.at[r, :], idx_v.at[b], sem_idx).wait()

                @pl.when(i * NBUF + b + 1 < ROWS_PER_TILE)
                def _prefetch():
                    pltpu.async_copy(
                        idx_hbm.at[r + 1, :], idx_v.at[1 - b], sem_idx)

                @pl.when(i * NBUF + b >= NBUF)
                def _drain():
                    pltpu.make_async_copy(
                        out_v.at[b], out_hbm.at[:, r - NBUF, :],
                        sem_out[b]).wait()

                @plsc.parallel_loop(0, SIDE // LANES, unroll=4)
                def gather_body(j):
                    iv = idx_v[b, pl.ds(j * LANES, LANES)]
                    for h in range(NHEADS):
                        off = iv + jnp.int32(h * NREL)
                        out_v[b, h, pl.ds(j * LANES, LANES)] = (
                            plsc.load_gather(table_v, [off]))

                pltpu.async_copy(
                    out_v.at[b], out_hbm.at[:, r, :], sem_out[b])
            return carry

        lax.fori_loop(0, ROWS_PER_TILE // NBUF, outer, 0)

        for b in range(NBUF):
            r = row0 + ROWS_PER_TILE - NBUF + b
            pltpu.make_async_copy(
                out_v.at[b], out_hbm.at[:, r, :], sem_out[b]).wait()

    return gather_kernel


_sc_gather = _make_sc_gather()


def kernel(relative_bias_table, relative_position_index):
    table_flat = relative_bias_table.reshape(-1)
    idx = relative_position_index.astype(jnp.int32)
    return _sc_gather(table_flat, idx)
